# baseline (speedup 1.0000x reference)
import jax
import jax.numpy as jnp
from jax import lax
from jax.experimental import pallas as pl
from jax.experimental.pallas import tpu as pltpu
from jax.experimental.pallas import tpu_sc as plsc

F32 = jnp.float32
I32 = jnp.int32

D = 128
L16 = 16
NC = 2
NS = 16
NW = NC * NS
CH = 128

N_NODES = 10000
NP = 10240
RPS = NP // NS
N_DUMMY = NP - N_NODES

N_EDGES = 320000
CHUNKS_E = 80
EPT_E = CHUNKS_E * CH
E_PAD = EPT_E * NW

N_EMB = N_NODES * 8
CHUNKS_G = 20
EPT_G = CHUNKS_G * CH
G_PAD = EPT_G * NW

QG = 8
NG = CHUNKS_E // QG


def _worker_id():
    c = lax.axis_index("c")
    s = lax.axis_index("s")
    return c, s, c * NS + s


def _zero_rows(rows_v):
    def body(i, carry):
        for k in range(D // L16):
            rows_v[i, pl.ds(k * L16, L16)] = jnp.zeros((L16,), F32)
        return carry
    lax.fori_loop(0, CH, body, 0)


def _zero_stripe_2d(zsrc_v, acc_s, s):
    for j in range(RPS // CH):
        pltpu.sync_copy(zsrc_v, acc_s.at[pl.ds(s * RPS + j * CH, CH)])


def _zero_stripe_1d(zsrc_row, acc_s, s):
    for j in range(RPS // CH):
        pltpu.sync_copy(zsrc_row, acc_s.at[pl.ds(s * RPS + j * CH, CH)])


def _pass1_body(gidx_hbm, nidx_hbm, esrc_hbm, edst_hbm, table_hbm,
                h0_out, dego_out, degi_out,
                gidx_v, nidx_v, srcA, srcB, dstA, dstB, rows0_v, rows1_v,
                ones_v,
                acc_s, dego_s, degi_s, sem0, sem1, dsem, isemA, isemB, isem):
    c, s, w = _worker_id()
    ebase = w * CHUNKS_E

    def ld_deg(gi, S, Dx, isem_):
        nb = ebase + gi * QG
        pltpu.async_copy(esrc_hbm.at[pl.ds(nb, QG)], S, isem_)
        pltpu.async_copy(edst_hbm.at[pl.ds(nb, QG)], Dx, isem_)

    cps = [pltpu.async_copy(gidx_hbm.at[pl.ds(w * CHUNKS_G, CHUNKS_G)], gidx_v, isem),
           pltpu.async_copy(nidx_hbm.at[pl.ds(w * CHUNKS_G, CHUNKS_G)], nidx_v, isem)]
    ld_deg(0, srcA, dstA, isemA)
    ld_deg(1, srcB, dstB, isemB)
    for k in range(CH // L16):
        ones_v[pl.ds(k * L16, L16)] = jnp.ones((L16,), F32)
    _zero_rows(rows0_v)
    _zero_stripe_2d(rows0_v, acc_s, s)
    _zero_stripe_1d(rows0_v.at[0], dego_s, s)
    _zero_stripe_1d(rows0_v.at[0], degi_s, s)
    for cp in cps:
        cp.wait()
    plsc.subcore_barrier()

    rows = [rows0_v, rows1_v]
    sems = [sem0, sem1]

    def g_start(i, b):
        pltpu.async_copy(table_hbm.at[gidx_v.at[i, 0]], rows[b], sems[b])

    def g_wait(b):
        pltpu.make_async_copy(table_hbm.at[pl.ds(0, CH)], rows[b], sems[b]).wait()

    def scat(i, b):
        pltpu.sync_copy(rows[b], acc_s.at[nidx_v.at[i, 0]], add=True)

    def deg_group(gi, S, Dx, isemP, isemQ):
        pltpu.make_async_copy(esrc_hbm.at[pl.ds(0, QG)], S, isemP).wait()
        pltpu.make_async_copy(esrc_hbm.at[pl.ds(0, QG)], Dx, isemP).wait()
        for j in range(QG):
            pltpu.async_copy(ones_v, dego_s.at[S.at[j, 0]], dsem, add=True)
            pltpu.async_copy(ones_v, degi_s.at[Dx.at[j, 0]], dsem, add=True)

    def deg_drain_reload(gi, S, Dx, isemP):
        def dr(i, carry):
            pltpu.make_async_copy(ones_v, dego_s.at[S.at[0, 0]], dsem).wait()
            pltpu.make_async_copy(ones_v, degi_s.at[Dx.at[0, 0]], dsem).wait()
            return carry
        lax.fori_loop(0, QG, dr, 0)

        @pl.when(gi + 2 < NG)
        def _():
            ld_deg(gi + 2, S, Dx, isemP)

    g_start(0, 0)

    def pair(g, carry):
        i0 = 2 * g
        g_start(i0 + 1, 1)
        degA = (g % 2) == 0

        @pl.when(degA)
        def _():
            deg_group(g, srcA, dstA, isemA, isemB)

        @pl.when(jnp.logical_not(degA))
        def _():
            deg_group(g, srcB, dstB, isemB, isemA)
        g_wait(0)
        scat(i0, 0)

        @pl.when(i0 + 2 < CHUNKS_G)
        def _():
            g_start(i0 + 2, 0)

        @pl.when(degA)
        def _():
            deg_drain_reload(g, srcA, dstA, isemA)

        @pl.when(jnp.logical_not(degA))
        def _():
            deg_drain_reload(g, srcB, dstB, isemB)
        g_wait(1)
        scat(i0 + 1, 1)
        return carry
    lax.fori_loop(0, CHUNKS_G // 2, pair, 0)

    plsc.subcore_barrier()
    row0 = c * NP + s * RPS
    pltpu.sync_copy(acc_s.at[pl.ds(s * RPS, RPS)], h0_out.at[pl.ds(row0, RPS)])
    pltpu.sync_copy(dego_s.at[pl.ds(s * RPS, RPS)], dego_out.at[pl.ds(row0, RPS)])
    pltpu.sync_copy(degi_s.at[pl.ds(s * RPS, RPS)], degi_out.at[pl.ds(row0, RPS)])


def _mp_pipeline(esrc_hbm, edst_hbm, p_hbm, acc_s,
                 srcA, srcB, dstA, dstB, rows, gsems, ssems, isems,
                 w, rin_hbm=None, cacc_s=None, rvs=None, vsems=None,
                 csems=None):
    base = w * CHUNKS_E
    with_c = rin_hbm is not None

    def ld_group(gi, S, Dx, isem):
        nb = base + gi * QG
        pltpu.async_copy(esrc_hbm.at[pl.ds(nb, QG)], S, isem)
        pltpu.async_copy(edst_hbm.at[pl.ds(nb, QG)], Dx, isem)

    def ld_wait(isem):
        pltpu.make_async_copy(esrc_hbm.at[pl.ds(0, QG)], srcA, isem).wait()
        pltpu.make_async_copy(esrc_hbm.at[pl.ds(0, QG)], dstA, isem).wait()

    def g_start(S, Dx, j, b):
        pltpu.async_copy(p_hbm.at[S.at[j, 0]], rows[b], gsems[b])
        if with_c:
            pltpu.async_copy(rin_hbm.at[Dx.at[j, 0]], rvs[b], vsems[b])

    def g_wait(b):
        pltpu.make_async_copy(p_hbm.at[pl.ds(0, CH)], rows[b], gsems[b]).wait()
        if with_c:
            pltpu.make_async_copy(rin_hbm.at[pl.ds(0, CH)], rvs[b], vsems[b]).wait()

    def s_start(S, Dx, j, b):
        pltpu.async_copy(rows[b], acc_s.at[Dx.at[j, 0]], ssems[b], add=True)
        if with_c:
            pltpu.async_copy(rvs[b], cacc_s.at[S.at[j, 0]], csems[b], add=True)

    def s_wait(b):
        pltpu.make_async_copy(rows[b], acc_s.at[pl.ds(0, CH)], ssems[b]).wait()
        if with_c:
            pltpu.make_async_copy(rvs[b], cacc_s.at[pl.ds(0, CH)], csems[b]).wait()

    def group(gi, S, Dx, isemP, So, Do, isemQ):
        for j in range(QG):
            b = j % 2
            if j + 1 < QG:
                if j == 0:
                    @pl.when(gi > 0)
                    def _():
                        s_wait(1 - b)
                else:
                    s_wait(1 - b)
                g_start(S, Dx, j + 1, 1 - b)
            else:
                @pl.when(gi + 1 < NG)
                def _():
                    s_wait(1 - b)
                    ld_wait(isemQ)
                    g_start(So, Do, 0, 1 - b)
            if j == 2:
                @pl.when(gi + 1 < NG)
                def _():
                    ld_group(gi + 1, So, Do, isemQ)
            g_wait(b)
            s_start(S, Dx, j, b)

    def pair(g, carry):
        group(2 * g, srcA, dstA, isems[0], srcB, dstB, isems[1])
        group(2 * g + 1, srcB, dstB, isems[1], srcA, dstA, isems[0])
        return carry

    def drain(_):
        s_wait(0)
        s_wait(1)

    ld_group(0, srcA, dstA, isems[0])
    return pair, drain


def _pass2_body(esrc_hbm, edst_hbm, p0_hbm, rin_hbm,
                agg_out, cacc_out,
                srcA, srcB, dstA, dstB, rows0_v, rows1_v, rv0_v, rv1_v, zb_v,
                acc_s, cacc_s,
                gsem0, gsem1, ssem0, ssem1, vsem0, vsem1, csem0, csem1,
                isemA, isemB):
    c, s, w = _worker_id()
    pair, drain = _mp_pipeline(esrc_hbm, edst_hbm, p0_hbm, acc_s,
                               srcA, srcB, dstA, dstB,
                               [rows0_v, rows1_v], [gsem0, gsem1],
                               [ssem0, ssem1], [isemA, isemB],
                               w, rin_hbm=rin_hbm, cacc_s=cacc_s,
                               rvs=[rv0_v, rv1_v], vsems=[vsem0, vsem1],
                               csems=[csem0, csem1])
    for k in range(CH // L16):
        zb_v[pl.ds(k * L16, L16)] = jnp.zeros((L16,), F32)
    _zero_rows(rows0_v)
    _zero_stripe_2d(rows0_v, acc_s, s)
    _zero_stripe_1d(zb_v, cacc_s, s)
    pltpu.make_async_copy(esrc_hbm.at[pl.ds(0, QG)], srcA, isemA).wait()
    pltpu.make_async_copy(esrc_hbm.at[pl.ds(0, QG)], dstA, isemA).wait()
    pltpu.async_copy(p0_hbm.at[srcA.at[0, 0]], rows0_v, gsem0)
    pltpu.async_copy(rin_hbm.at[dstA.at[0, 0]], rv0_v, vsem0)
    plsc.subcore_barrier()
    lax.fori_loop(0, NG // 2, pair, 0)
    drain(0)
    plsc.subcore_barrier()
    row0 = c * NP + s * RPS
    pltpu.sync_copy(acc_s.at[pl.ds(s * RPS, RPS)], agg_out.at[pl.ds(row0, RPS)])
    pltpu.sync_copy(cacc_s.at[pl.ds(s * RPS, RPS)], cacc_out.at[pl.ds(row0, RPS)])


def _pass3_body(esrc_hbm, edst_hbm, p1_hbm,
                agg_out,
                srcA, srcB, dstA, dstB, rows0_v, rows1_v,
                acc_s,
                gsem0, gsem1, ssem0, ssem1, isemA, isemB):
    c, s, w = _worker_id()
    pair, drain = _mp_pipeline(esrc_hbm, edst_hbm, p1_hbm, acc_s,
                               srcA, srcB, dstA, dstB,
                               [rows0_v, rows1_v], [gsem0, gsem1],
                               [ssem0, ssem1], [isemA, isemB], w)
    _zero_rows(rows0_v)
    _zero_stripe_2d(rows0_v, acc_s, s)
    pltpu.make_async_copy(esrc_hbm.at[pl.ds(0, QG)], srcA, isemA).wait()
    pltpu.make_async_copy(esrc_hbm.at[pl.ds(0, QG)], dstA, isemA).wait()
    pltpu.async_copy(p1_hbm.at[srcA.at[0, 0]], rows0_v, gsem0)
    plsc.subcore_barrier()
    lax.fori_loop(0, NG // 2, pair, 0)
    drain(0)
    plsc.subcore_barrier()
    row0 = c * NP + s * RPS
    pltpu.sync_copy(acc_s.at[pl.ds(s * RPS, RPS)], agg_out.at[pl.ds(row0, RPS)])


def _sc_mesh():
    return plsc.VectorSubcoreMesh(core_axis_name="c", subcore_axis_name="s",
                                  num_cores=NC, num_subcores=NS)


def _tc_a_body(h0p, degop, degip, p0_ref, rin_ref, rout_ref):
    dego = degop[...][:NP] + degop[...][NP:]
    degi = degip[...][:NP] + degip[...][NP:]
    rout = lax.rsqrt(jnp.maximum(dego, 1.0))
    rin = lax.rsqrt(jnp.maximum(degi, 1.0))
    rin_ref[...] = rin
    rout_ref[...] = rout
    h0 = h0p[...][:NP] + h0p[...][NP:]
    p0_ref[...] = h0 * rout[:, None]


def _tc_b_body(aggp, rin, rout, p1_ref):
    agg = aggp[...][:NP] + aggp[...][NP:]
    h1 = jnp.maximum(agg * rin[...][:, None], 0.0)
    p1_ref[...] = h1 * rout[...][:, None]


def _tc_c_body(aggp, rin, rout, caccp, w1, w2, out_ref):
    agg = aggp[...][:NP] + aggp[...][NP:]
    a1 = agg * rin[...][:, None]
    h2 = jnp.maximum(jnp.dot(a1, w1[...], preferred_element_type=F32), 0.0)
    cc = (caccp[...][:NP] + caccp[...][NP:]) * rout[...]
    rid = lax.broadcasted_iota(I32, (NP,), 0)
    wvec = jnp.where(rid < N_NODES, cc, 0.0)
    s = jnp.sum(h2 * wvec[:, None], axis=0, keepdims=True)
    out_ref[...] = jnp.dot(s, w2[...], preferred_element_type=F32)


def kernel(feature, edge_index, table, W1, W2):
    n = feature.shape[0]
    nf = feature.shape[1]
    src = edge_index[0].astype(I32)
    dst = edge_index[1].astype(I32)

    e_pad = E_PAD - src.shape[0]
    pad_rows = N_NODES + (jnp.arange(e_pad, dtype=I32) % N_DUMMY)
    srcp = jnp.concatenate([src, pad_rows]).reshape(NW * CHUNKS_E, 1, CH)
    dstp = jnp.concatenate([dst, pad_rows]).reshape(NW * CHUNKS_E, 1, CH)

    gidx = feature.reshape(-1).astype(I32)
    nidx = jnp.repeat(jnp.arange(n, dtype=I32), nf)
    g_pad = G_PAD - gidx.shape[0]
    gpad_rows = jnp.arange(g_pad, dtype=I32) % jnp.int32(table.shape[0])
    npad_rows = N_NODES + (jnp.arange(g_pad, dtype=I32) % N_DUMMY)
    gidxp = jnp.concatenate([gidx, gpad_rows]).reshape(NW * CHUNKS_G, 1, CH)
    nidxp = jnp.concatenate([nidx, npad_rows]).reshape(NW * CHUNKS_G, 1, CH)

    mesh = _sc_mesh()

    pass1 = pl.kernel(
        _pass1_body, mesh=mesh,
        out_type=[jax.ShapeDtypeStruct((2 * NP, D), F32),
                  jax.ShapeDtypeStruct((2 * NP,), F32),
                  jax.ShapeDtypeStruct((2 * NP,), F32)],
        scratch_types=[
            pltpu.VMEM((CHUNKS_G, 1, CH), I32),
            pltpu.VMEM((CHUNKS_G, 1, CH), I32),
            pltpu.VMEM((QG, 1, CH), I32),
            pltpu.VMEM((QG, 1, CH), I32),
            pltpu.VMEM((QG, 1, CH), I32),
            pltpu.VMEM((QG, 1, CH), I32),
            pltpu.VMEM((CH, D), F32),
            pltpu.VMEM((CH, D), F32),
            pltpu.VMEM((CH,), F32),
            pltpu.VMEM_SHARED((NP, D), F32),
            pltpu.VMEM_SHARED((NP,), F32),
            pltpu.VMEM_SHARED((NP,), F32),
            pltpu.SemaphoreType.DMA,
            pltpu.SemaphoreType.DMA,
            pltpu.SemaphoreType.DMA,
            pltpu.SemaphoreType.DMA,
            pltpu.SemaphoreType.DMA,
            pltpu.SemaphoreType.DMA,
        ])
    h0p, degop, degip = pass1(gidxp, nidxp, srcp, dstp, table)

    tc_a = pl.pallas_call(
        _tc_a_body,
        out_shape=[jax.ShapeDtypeStruct((NP, D), F32),
                   jax.ShapeDtypeStruct((NP,), F32),
                   jax.ShapeDtypeStruct((NP,), F32)])
    p0, rin, rout = tc_a(h0p, degop, degip)

    pass2 = pl.kernel(
        _pass2_body, mesh=mesh,
        out_type=[jax.ShapeDtypeStruct((2 * NP, D), F32),
                  jax.ShapeDtypeStruct((2 * NP,), F32)],
        scratch_types=[
            pltpu.VMEM((QG, 1, CH), I32),
            pltpu.VMEM((QG, 1, CH), I32),
            pltpu.VMEM((QG, 1, CH), I32),
            pltpu.VMEM((QG, 1, CH), I32),
            pltpu.VMEM((CH, D), F32),
            pltpu.VMEM((CH, D), F32),
            pltpu.VMEM((CH,), F32),
            pltpu.VMEM((CH,), F32),
            pltpu.VMEM((CH,), F32),
            pltpu.VMEM_SHARED((NP, D), F32),
            pltpu.VMEM_SHARED((NP,), F32),
            pltpu.SemaphoreType.DMA,
            pltpu.SemaphoreType.DMA,
            pltpu.SemaphoreType.DMA,
            pltpu.SemaphoreType.DMA,
            pltpu.SemaphoreType.DMA,
            pltpu.SemaphoreType.DMA,
            pltpu.SemaphoreType.DMA,
            pltpu.SemaphoreType.DMA,
            pltpu.SemaphoreType.DMA,
            pltpu.SemaphoreType.DMA,
        ])
    agg0p, caccp = pass2(srcp, dstp, p0, rin)

    tc_b = pl.pallas_call(
        _tc_b_body,
        out_shape=jax.ShapeDtypeStruct((NP, D), F32))
    p1 = tc_b(agg0p, rin, rout)

    pass3 = pl.kernel(
        _pass3_body, mesh=mesh,
        out_type=jax.ShapeDtypeStruct((2 * NP, D), F32),
        scratch_types=[
            pltpu.VMEM((QG, 1, CH), I32),
            pltpu.VMEM((QG, 1, CH), I32),
            pltpu.VMEM((QG, 1, CH), I32),
            pltpu.VMEM((QG, 1, CH), I32),
            pltpu.VMEM((CH, D), F32),
            pltpu.VMEM((CH, D), F32),
            pltpu.VMEM_SHARED((NP, D), F32),
            pltpu.SemaphoreType.DMA,
            pltpu.SemaphoreType.DMA,
            pltpu.SemaphoreType.DMA,
            pltpu.SemaphoreType.DMA,
            pltpu.SemaphoreType.DMA,
            pltpu.SemaphoreType.DMA,
        ])
    agg1p = pass3(srcp, dstp, p1)

    tc_c = pl.pallas_call(
        _tc_c_body,
        out_shape=jax.ShapeDtypeStruct((1, D), F32))
    out = tc_c(agg1p, rin, rout, caccp, W1, W2)
    return out.reshape(D)


# speedup vs baseline: 1.0035x; 1.0035x over previous
import jax
import jax.numpy as jnp
from jax import lax
from jax.experimental import pallas as pl
from jax.experimental.pallas import tpu as pltpu
from jax.experimental.pallas import tpu_sc as plsc

F32 = jnp.float32
I32 = jnp.int32

D = 128
L16 = 16
NC = 2
NS = 16
NW = NC * NS
CH = 128

N_NODES = 10000
NP = 10240
RPS = NP // NS
N_DUMMY = NP - N_NODES

N_EDGES = 320000
CHUNKS_E = 80
EPT_E = CHUNKS_E * CH
E_PAD = EPT_E * NW

N_EMB = N_NODES * 8
CHUNKS_G = 20
EPT_G = CHUNKS_G * CH
G_PAD = EPT_G * NW

QG = 8
NG = CHUNKS_E // QG


def _worker_id():
    c = lax.axis_index("c")
    s = lax.axis_index("s")
    return c, s, c * NS + s


def _zero_rows(rows_v):
    def body(i, carry):
        for k in range(D // L16):
            rows_v[i, pl.ds(k * L16, L16)] = jnp.zeros((L16,), F32)
        return carry
    lax.fori_loop(0, CH, body, 0)


def _zero_stripe_2d(zsrc_v, acc_s, s):
    for j in range(RPS // CH):
        pltpu.sync_copy(zsrc_v, acc_s.at[pl.ds(s * RPS + j * CH, CH)])


def _zero_stripe_1d(zsrc_row, acc_s, s):
    for j in range(RPS // CH):
        pltpu.sync_copy(zsrc_row, acc_s.at[pl.ds(s * RPS + j * CH, CH)])


def _pass1_body(gidx_hbm, nidx_hbm, esrc_hbm, edst_hbm, table_hbm,
                h0_out, dego_out, degi_out,
                gidx_v, nidx_v, srcA, srcB, dstA, dstB, rows0_v, rows1_v,
                ones_v,
                acc_s, dego_s, degi_s, sem0, sem1, dsem, isemA, isemB, isem):
    c, s, w = _worker_id()
    ebase = w * CHUNKS_E

    def ld_deg(gi, S, Dx, isem_):
        nb = ebase + gi * QG
        pltpu.async_copy(esrc_hbm.at[pl.ds(nb, QG)], S, isem_)
        pltpu.async_copy(edst_hbm.at[pl.ds(nb, QG)], Dx, isem_)

    cps = [pltpu.async_copy(gidx_hbm.at[pl.ds(w * CHUNKS_G, CHUNKS_G)], gidx_v, isem),
           pltpu.async_copy(nidx_hbm.at[pl.ds(w * CHUNKS_G, CHUNKS_G)], nidx_v, isem)]
    ld_deg(0, srcA, dstA, isemA)
    ld_deg(1, srcB, dstB, isemB)
    for k in range(CH // L16):
        ones_v[pl.ds(k * L16, L16)] = jnp.ones((L16,), F32)
    _zero_rows(rows0_v)
    _zero_stripe_2d(rows0_v, acc_s, s)
    _zero_stripe_1d(rows0_v.at[0], dego_s, s)
    _zero_stripe_1d(rows0_v.at[0], degi_s, s)
    for cp in cps:
        cp.wait()
    plsc.subcore_barrier()

    rows = [rows0_v, rows1_v]
    sems = [sem0, sem1]

    def g_start(i, b):
        pltpu.async_copy(table_hbm.at[gidx_v.at[i, 0]], rows[b], sems[b])

    def g_wait(b):
        pltpu.make_async_copy(table_hbm.at[pl.ds(0, CH)], rows[b], sems[b]).wait()

    def scat(i, b):
        pltpu.sync_copy(rows[b], acc_s.at[nidx_v.at[i, 0]], add=True)

    def deg_group(gi, S, Dx, isemP, isemQ):
        pltpu.make_async_copy(esrc_hbm.at[pl.ds(0, QG)], S, isemP).wait()
        pltpu.make_async_copy(esrc_hbm.at[pl.ds(0, QG)], Dx, isemP).wait()
        for j in range(QG):
            pltpu.async_copy(ones_v, dego_s.at[S.at[j, 0]], dsem, add=True)
            pltpu.async_copy(ones_v, degi_s.at[Dx.at[j, 0]], dsem, add=True)

    def deg_drain_reload(gi, S, Dx, isemP):
        def dr(i, carry):
            pltpu.make_async_copy(ones_v, dego_s.at[S.at[0, 0]], dsem).wait()
            pltpu.make_async_copy(ones_v, degi_s.at[Dx.at[0, 0]], dsem).wait()
            return carry
        lax.fori_loop(0, QG, dr, 0)

        @pl.when(gi + 2 < NG)
        def _():
            ld_deg(gi + 2, S, Dx, isemP)

    g_start(0, 0)

    def pair(g, carry):
        i0 = 2 * g
        g_start(i0 + 1, 1)
        degA = (g % 2) == 0

        @pl.when(degA)
        def _():
            deg_group(g, srcA, dstA, isemA, isemB)

        @pl.when(jnp.logical_not(degA))
        def _():
            deg_group(g, srcB, dstB, isemB, isemA)
        g_wait(0)
        scat(i0, 0)

        @pl.when(i0 + 2 < CHUNKS_G)
        def _():
            g_start(i0 + 2, 0)

        @pl.when(degA)
        def _():
            deg_drain_reload(g, srcA, dstA, isemA)

        @pl.when(jnp.logical_not(degA))
        def _():
            deg_drain_reload(g, srcB, dstB, isemB)
        g_wait(1)
        scat(i0 + 1, 1)
        return carry
    lax.fori_loop(0, CHUNKS_G // 2, pair, 0)

    plsc.subcore_barrier()
    row0 = c * NP + s * RPS
    pltpu.sync_copy(acc_s.at[pl.ds(s * RPS, RPS)], h0_out.at[pl.ds(row0, RPS)])
    pltpu.sync_copy(dego_s.at[pl.ds(s * RPS, RPS)], dego_out.at[pl.ds(row0, RPS)])
    pltpu.sync_copy(degi_s.at[pl.ds(s * RPS, RPS)], degi_out.at[pl.ds(row0, RPS)])


def _mp_pipeline(esrc_hbm, edst_hbm, p_hbm, acc_s,
                 srcA, srcB, dstA, dstB, rows, gsems, isems,
                 w, rin_hbm=None, cacc_s=None, rvs=None, vsems=None):
    base = w * CHUNKS_E
    with_c = rin_hbm is not None

    def ld_group(gi, S, Dx, isem):
        nb = base + gi * QG
        pltpu.async_copy(esrc_hbm.at[pl.ds(nb, QG)], S, isem)
        pltpu.async_copy(edst_hbm.at[pl.ds(nb, QG)], Dx, isem)

    def ld_wait(isem):
        pltpu.make_async_copy(esrc_hbm.at[pl.ds(0, QG)], srcA, isem).wait()
        pltpu.make_async_copy(esrc_hbm.at[pl.ds(0, QG)], dstA, isem).wait()

    def g_start(S, Dx, j, b):
        pltpu.async_copy(p_hbm.at[S.at[j, 0]], rows[b], gsems[b])
        if with_c:
            pltpu.async_copy(rin_hbm.at[Dx.at[j, 0]], rvs[b], vsems[b])

    def g_wait(b):
        pltpu.make_async_copy(p_hbm.at[pl.ds(0, CH)], rows[b], gsems[b]).wait()
        if with_c:
            pltpu.make_async_copy(rin_hbm.at[pl.ds(0, CH)], rvs[b], vsems[b]).wait()

    def scat(S, Dx, j, b):
        pltpu.sync_copy(rows[b], acc_s.at[Dx.at[j, 0]], add=True)
        if with_c:
            pltpu.sync_copy(rvs[b], cacc_s.at[S.at[j, 0]], add=True)

    def group(gi, S, Dx, isemP, So, Do, isemQ):
        for j in range(QG):
            b = j % 2
            if j + 1 < QG:
                g_start(S, Dx, j + 1, 1 - b)
            else:
                @pl.when(gi + 1 < NG)
                def _():
                    ld_wait(isemQ)
                    g_start(So, Do, 0, 1 - b)
            g_wait(b)
            scat(S, Dx, j, b)

        @pl.when(gi + 2 < NG)
        def _():
            ld_group(gi + 2, S, Dx, isemP)

    def pair(g, carry):
        group(2 * g, srcA, dstA, isems[0], srcB, dstB, isems[1])
        group(2 * g + 1, srcB, dstB, isems[1], srcA, dstA, isems[0])
        return carry

    ld_group(0, srcA, dstA, isems[0])
    ld_group(1, srcB, dstB, isems[1])
    return pair


def _pass2_body(esrc_hbm, edst_hbm, p0_hbm, rin_hbm,
                agg_out, cacc_out,
                srcA, srcB, dstA, dstB, rows0_v, rows1_v, rv0_v, rv1_v, zb_v,
                acc_s, cacc_s,
                gsem0, gsem1, vsem0, vsem1, isemA, isemB):
    c, s, w = _worker_id()
    pair = _mp_pipeline(esrc_hbm, edst_hbm, p0_hbm, acc_s,
                        srcA, srcB, dstA, dstB,
                        [rows0_v, rows1_v], [gsem0, gsem1], [isemA, isemB],
                        w, rin_hbm=rin_hbm, cacc_s=cacc_s,
                        rvs=[rv0_v, rv1_v], vsems=[vsem0, vsem1])
    for k in range(CH // L16):
        zb_v[pl.ds(k * L16, L16)] = jnp.zeros((L16,), F32)
    _zero_rows(rows0_v)
    _zero_stripe_2d(rows0_v, acc_s, s)
    _zero_stripe_1d(zb_v, cacc_s, s)
    pltpu.make_async_copy(esrc_hbm.at[pl.ds(0, QG)], srcA, isemA).wait()
    pltpu.make_async_copy(esrc_hbm.at[pl.ds(0, QG)], dstA, isemA).wait()
    pltpu.async_copy(p0_hbm.at[srcA.at[0, 0]], rows0_v, gsem0)
    pltpu.async_copy(rin_hbm.at[dstA.at[0, 0]], rv0_v, vsem0)
    plsc.subcore_barrier()
    lax.fori_loop(0, NG // 2, pair, 0)
    plsc.subcore_barrier()
    row0 = c * NP + s * RPS
    pltpu.sync_copy(acc_s.at[pl.ds(s * RPS, RPS)], agg_out.at[pl.ds(row0, RPS)])
    pltpu.sync_copy(cacc_s.at[pl.ds(s * RPS, RPS)], cacc_out.at[pl.ds(row0, RPS)])


def _pass3_body(esrc_hbm, edst_hbm, p1_hbm,
                agg_out,
                srcA, srcB, dstA, dstB, rows0_v, rows1_v,
                acc_s,
                gsem0, gsem1, isemA, isemB):
    c, s, w = _worker_id()
    pair = _mp_pipeline(esrc_hbm, edst_hbm, p1_hbm, acc_s,
                        srcA, srcB, dstA, dstB,
                        [rows0_v, rows1_v], [gsem0, gsem1], [isemA, isemB], w)
    _zero_rows(rows0_v)
    _zero_stripe_2d(rows0_v, acc_s, s)
    pltpu.make_async_copy(esrc_hbm.at[pl.ds(0, QG)], srcA, isemA).wait()
    pltpu.make_async_copy(esrc_hbm.at[pl.ds(0, QG)], dstA, isemA).wait()
    pltpu.async_copy(p1_hbm.at[srcA.at[0, 0]], rows0_v, gsem0)
    plsc.subcore_barrier()
    lax.fori_loop(0, NG // 2, pair, 0)
    plsc.subcore_barrier()
    row0 = c * NP + s * RPS
    pltpu.sync_copy(acc_s.at[pl.ds(s * RPS, RPS)], agg_out.at[pl.ds(row0, RPS)])


def _sc_mesh():
    return plsc.VectorSubcoreMesh(core_axis_name="c", subcore_axis_name="s",
                                  num_cores=NC, num_subcores=NS)


def _tc_a_body(h0p, degop, degip, p0_ref, rin_ref, rout_ref):
    dego = degop[...][:NP] + degop[...][NP:]
    degi = degip[...][:NP] + degip[...][NP:]
    rout = lax.rsqrt(jnp.maximum(dego, 1.0))
    rin = lax.rsqrt(jnp.maximum(degi, 1.0))
    rin_ref[...] = rin
    rout_ref[...] = rout
    h0 = h0p[...][:NP] + h0p[...][NP:]
    p0_ref[...] = h0 * rout[:, None]


def _tc_b_body(aggp, rin, rout, p1_ref):
    agg = aggp[...][:NP] + aggp[...][NP:]
    h1 = jnp.maximum(agg * rin[...][:, None], 0.0)
    p1_ref[...] = h1 * rout[...][:, None]


def _tc_c_body(aggp, rin, rout, caccp, w1, w2, out_ref):
    agg = aggp[...][:NP] + aggp[...][NP:]
    a1 = agg * rin[...][:, None]
    h2 = jnp.maximum(jnp.dot(a1, w1[...], preferred_element_type=F32), 0.0)
    cc = (caccp[...][:NP] + caccp[...][NP:]) * rout[...]
    rid = lax.broadcasted_iota(I32, (NP,), 0)
    wvec = jnp.where(rid < N_NODES, cc, 0.0)
    s = jnp.sum(h2 * wvec[:, None], axis=0, keepdims=True)
    out_ref[...] = jnp.dot(s, w2[...], preferred_element_type=F32)


def kernel(feature, edge_index, table, W1, W2):
    n = feature.shape[0]
    nf = feature.shape[1]
    src = edge_index[0].astype(I32)
    dst = edge_index[1].astype(I32)

    e_pad = E_PAD - src.shape[0]
    pad_rows = N_NODES + (jnp.arange(e_pad, dtype=I32) % N_DUMMY)
    srcp = jnp.concatenate([src, pad_rows]).reshape(NW * CHUNKS_E, 1, CH)
    dstp = jnp.concatenate([dst, pad_rows]).reshape(NW * CHUNKS_E, 1, CH)

    gidx = feature.reshape(-1).astype(I32)
    nidx = jnp.repeat(jnp.arange(n, dtype=I32), nf)
    g_pad = G_PAD - gidx.shape[0]
    gpad_rows = jnp.arange(g_pad, dtype=I32) % jnp.int32(table.shape[0])
    npad_rows = N_NODES + (jnp.arange(g_pad, dtype=I32) % N_DUMMY)
    gidxp = jnp.concatenate([gidx, gpad_rows]).reshape(NW * CHUNKS_G, 1, CH)
    nidxp = jnp.concatenate([nidx, npad_rows]).reshape(NW * CHUNKS_G, 1, CH)

    mesh = _sc_mesh()

    pass1 = pl.kernel(
        _pass1_body, mesh=mesh,
        out_type=[jax.ShapeDtypeStruct((2 * NP, D), F32),
                  jax.ShapeDtypeStruct((2 * NP,), F32),
                  jax.ShapeDtypeStruct((2 * NP,), F32)],
        scratch_types=[
            pltpu.VMEM((CHUNKS_G, 1, CH), I32),
            pltpu.VMEM((CHUNKS_G, 1, CH), I32),
            pltpu.VMEM((QG, 1, CH), I32),
            pltpu.VMEM((QG, 1, CH), I32),
            pltpu.VMEM((QG, 1, CH), I32),
            pltpu.VMEM((QG, 1, CH), I32),
            pltpu.VMEM((CH, D), F32),
            pltpu.VMEM((CH, D), F32),
            pltpu.VMEM((CH,), F32),
            pltpu.VMEM_SHARED((NP, D), F32),
            pltpu.VMEM_SHARED((NP,), F32),
            pltpu.VMEM_SHARED((NP,), F32),
            pltpu.SemaphoreType.DMA,
            pltpu.SemaphoreType.DMA,
            pltpu.SemaphoreType.DMA,
            pltpu.SemaphoreType.DMA,
            pltpu.SemaphoreType.DMA,
            pltpu.SemaphoreType.DMA,
        ])
    h0p, degop, degip = pass1(gidxp, nidxp, srcp, dstp, table)

    tc_a = pl.pallas_call(
        _tc_a_body,
        out_shape=[jax.ShapeDtypeStruct((NP, D), F32),
                   jax.ShapeDtypeStruct((NP,), F32),
                   jax.ShapeDtypeStruct((NP,), F32)])
    p0, rin, rout = tc_a(h0p, degop, degip)

    pass2 = pl.kernel(
        _pass2_body, mesh=mesh,
        out_type=[jax.ShapeDtypeStruct((2 * NP, D), F32),
                  jax.ShapeDtypeStruct((2 * NP,), F32)],
        scratch_types=[
            pltpu.VMEM((QG, 1, CH), I32),
            pltpu.VMEM((QG, 1, CH), I32),
            pltpu.VMEM((QG, 1, CH), I32),
            pltpu.VMEM((QG, 1, CH), I32),
            pltpu.VMEM((CH, D), F32),
            pltpu.VMEM((CH, D), F32),
            pltpu.VMEM((CH,), F32),
            pltpu.VMEM((CH,), F32),
            pltpu.VMEM((CH,), F32),
            pltpu.VMEM_SHARED((NP, D), F32),
            pltpu.VMEM_SHARED((NP,), F32),
            pltpu.SemaphoreType.DMA,
            pltpu.SemaphoreType.DMA,
            pltpu.SemaphoreType.DMA,
            pltpu.SemaphoreType.DMA,
            pltpu.SemaphoreType.DMA,
            pltpu.SemaphoreType.DMA,
        ])
    agg0p, caccp = pass2(srcp, dstp, p0, rin)

    tc_b = pl.pallas_call(
        _tc_b_body,
        out_shape=jax.ShapeDtypeStruct((NP, D), F32))
    p1 = tc_b(agg0p, rin, rout)

    pass3 = pl.kernel(
        _pass3_body, mesh=mesh,
        out_type=jax.ShapeDtypeStruct((2 * NP, D), F32),
        scratch_types=[
            pltpu.VMEM((QG, 1, CH), I32),
            pltpu.VMEM((QG, 1, CH), I32),
            pltpu.VMEM((QG, 1, CH), I32),
            pltpu.VMEM((QG, 1, CH), I32),
            pltpu.VMEM((CH, D), F32),
            pltpu.VMEM((CH, D), F32),
            pltpu.VMEM_SHARED((NP, D), F32),
            pltpu.SemaphoreType.DMA,
            pltpu.SemaphoreType.DMA,
            pltpu.SemaphoreType.DMA,
            pltpu.SemaphoreType.DMA,
        ])
    agg1p = pass3(srcp, dstp, p1)

    tc_c = pl.pallas_call(
        _tc_c_body,
        out_shape=jax.ShapeDtypeStruct((1, D), F32))
    out = tc_c(agg1p, rin, rout, caccp, W1, W2)
    return out.reshape(D)


# speedup vs baseline: 1.0282x; 1.0246x over previous
import jax
import jax.numpy as jnp
from jax import lax
from jax.experimental import pallas as pl
from jax.experimental.pallas import tpu as pltpu
from jax.experimental.pallas import tpu_sc as plsc

F32 = jnp.float32
I32 = jnp.int32

D = 128
L16 = 16
NC = 2
NS = 16
NW = NC * NS
CH = 128

N_NODES = 10000
NP = 10240
RPS = NP // NS
N_DUMMY = NP - N_NODES

N_EDGES = 320000
CHUNKS_E = 80
EPT_E = CHUNKS_E * CH
E_PAD = EPT_E * NW

N_EMB = N_NODES * 8
CHUNKS_G = 20
EPT_G = CHUNKS_G * CH
G_PAD = EPT_G * NW

QG = 8
NG = CHUNKS_E // QG


def _worker_id():
    c = lax.axis_index("c")
    s = lax.axis_index("s")
    return c, s, c * NS + s


def _zero_rows(rows_v):
    def body(i, carry):
        for k in range(D // L16):
            rows_v[i, pl.ds(k * L16, L16)] = jnp.zeros((L16,), F32)
        return carry
    lax.fori_loop(0, CH, body, 0)


def _zero_stripe_2d(zsrc_v, acc_s, s):
    for j in range(RPS // CH):
        pltpu.sync_copy(zsrc_v, acc_s.at[pl.ds(s * RPS + j * CH, CH)])


def _zero_stripe_1d(zsrc_row, acc_s, s):
    for j in range(RPS // CH):
        pltpu.sync_copy(zsrc_row, acc_s.at[pl.ds(s * RPS + j * CH, CH)])


def _pass1_body(gidx_hbm, nidx_hbm, esrc_hbm, edst_hbm, table_hbm,
                h0_out, dego_out, degi_out,
                gidx_v, nidx_v, srcA, srcB, dstA, dstB, rows0_v, rows1_v,
                ones_v,
                acc_s, dego_s, degi_s, sem0, sem1, dsem, isemA, isemB, isem):
    c, s, w = _worker_id()
    ebase = w * CHUNKS_E

    def ld_deg(gi, S, Dx, isem_):
        nb = ebase + gi * QG
        pltpu.async_copy(esrc_hbm.at[pl.ds(nb, QG)], S, isem_)
        pltpu.async_copy(edst_hbm.at[pl.ds(nb, QG)], Dx, isem_)

    cps = [pltpu.async_copy(gidx_hbm.at[pl.ds(w * CHUNKS_G, CHUNKS_G)], gidx_v, isem),
           pltpu.async_copy(nidx_hbm.at[pl.ds(w * CHUNKS_G, CHUNKS_G)], nidx_v, isem)]
    ld_deg(0, srcA, dstA, isemA)
    ld_deg(1, srcB, dstB, isemB)
    for k in range(CH // L16):
        ones_v[pl.ds(k * L16, L16)] = jnp.ones((L16,), F32)
    _zero_rows(rows0_v)
    _zero_stripe_2d(rows0_v, acc_s, s)
    _zero_stripe_1d(rows0_v.at[0], dego_s, s)
    _zero_stripe_1d(rows0_v.at[0], degi_s, s)
    for cp in cps:
        cp.wait()
    plsc.subcore_barrier()

    rows = [rows0_v, rows1_v]
    sems = [sem0, sem1]

    def g_start(i, b):
        pltpu.async_copy(table_hbm.at[gidx_v.at[i, 0]], rows[b], sems[b])

    def g_wait(b):
        pltpu.make_async_copy(table_hbm.at[pl.ds(0, CH)], rows[b], sems[b]).wait()

    def scat(i, b):
        pltpu.sync_copy(rows[b], acc_s.at[nidx_v.at[i, 0]], add=True)

    def deg_group(gi, S, Dx, isemP, isemQ):
        pltpu.make_async_copy(esrc_hbm.at[pl.ds(0, QG)], S, isemP).wait()
        pltpu.make_async_copy(esrc_hbm.at[pl.ds(0, QG)], Dx, isemP).wait()
        for j in range(QG):
            pltpu.async_copy(ones_v, dego_s.at[S.at[j, 0]], dsem, add=True)
            pltpu.async_copy(ones_v, degi_s.at[Dx.at[j, 0]], dsem, add=True)

    def deg_drain_reload(gi, S, Dx, isemP):
        def dr(i, carry):
            pltpu.make_async_copy(ones_v, dego_s.at[S.at[0, 0]], dsem).wait()
            pltpu.make_async_copy(ones_v, degi_s.at[Dx.at[0, 0]], dsem).wait()
            return carry
        lax.fori_loop(0, QG, dr, 0)

        @pl.when(gi + 2 < NG)
        def _():
            ld_deg(gi + 2, S, Dx, isemP)

    g_start(0, 0)

    def pair(g, carry):
        i0 = 2 * g
        g_start(i0 + 1, 1)
        degA = (g % 2) == 0

        @pl.when(degA)
        def _():
            deg_group(g, srcA, dstA, isemA, isemB)

        @pl.when(jnp.logical_not(degA))
        def _():
            deg_group(g, srcB, dstB, isemB, isemA)
        g_wait(0)
        scat(i0, 0)

        @pl.when(i0 + 2 < CHUNKS_G)
        def _():
            g_start(i0 + 2, 0)

        @pl.when(degA)
        def _():
            deg_drain_reload(g, srcA, dstA, isemA)

        @pl.when(jnp.logical_not(degA))
        def _():
            deg_drain_reload(g, srcB, dstB, isemB)
        g_wait(1)
        scat(i0 + 1, 1)
        return carry
    lax.fori_loop(0, CHUNKS_G // 2, pair, 0)

    plsc.subcore_barrier()
    row0 = c * NP + s * RPS
    hrow = c * (NP // 2) + s * (RPS // 2)
    pltpu.sync_copy(acc_s.at[pl.ds(hrow, RPS // 2)], h0_out.at[pl.ds(hrow, RPS // 2)])
    pltpu.sync_copy(dego_s.at[pl.ds(s * RPS, RPS)], dego_out.at[pl.ds(row0, RPS)])
    pltpu.sync_copy(degi_s.at[pl.ds(s * RPS, RPS)], degi_out.at[pl.ds(row0, RPS)])


def _mp_pipeline(esrc_hbm, edst_hbm, p_hbm, acc_s,
                 srcA, srcB, dstA, dstB, rows, gsems, isems,
                 w, rin_s=None, cacc_s=None, rvs=None, vsems=None):
    base = w * CHUNKS_E
    with_c = rin_s is not None

    def ld_group(gi, S, Dx, isem):
        nb = base + gi * QG
        pltpu.async_copy(esrc_hbm.at[pl.ds(nb, QG)], S, isem)
        pltpu.async_copy(edst_hbm.at[pl.ds(nb, QG)], Dx, isem)

    def ld_wait(isem):
        pltpu.make_async_copy(esrc_hbm.at[pl.ds(0, QG)], srcA, isem).wait()
        pltpu.make_async_copy(esrc_hbm.at[pl.ds(0, QG)], dstA, isem).wait()

    def g_start(S, Dx, j, b):
        pltpu.async_copy(p_hbm.at[S.at[j, 0]], rows[b], gsems[b])
        if with_c:
            pltpu.async_copy(rin_s.at[Dx.at[j, 0]], rvs[b], vsems[b])

    def g_wait(b):
        pltpu.make_async_copy(p_hbm.at[pl.ds(0, CH)], rows[b], gsems[b]).wait()
        if with_c:
            pltpu.make_async_copy(rin_s.at[pl.ds(0, CH)], rvs[b], vsems[b]).wait()

    def scat(S, Dx, j, b):
        pltpu.sync_copy(rows[b], acc_s.at[Dx.at[j, 0]], add=True)
        if with_c:
            pltpu.sync_copy(rvs[b], cacc_s.at[S.at[j, 0]], add=True)

    def group(gi, S, Dx, isemP, So, Do, isemQ):
        for j in range(QG):
            b = j % 2
            if j + 1 < QG:
                g_start(S, Dx, j + 1, 1 - b)
            else:
                @pl.when(gi + 1 < NG)
                def _():
                    ld_wait(isemQ)
                    g_start(So, Do, 0, 1 - b)
            g_wait(b)
            scat(S, Dx, j, b)

        @pl.when(gi + 2 < NG)
        def _():
            ld_group(gi + 2, S, Dx, isemP)

    def pair(g, carry):
        group(2 * g, srcA, dstA, isems[0], srcB, dstB, isems[1])
        group(2 * g + 1, srcB, dstB, isems[1], srcA, dstA, isems[0])
        return carry

    ld_group(0, srcA, dstA, isems[0])
    ld_group(1, srcB, dstB, isems[1])
    return pair


def _pass2_body(esrc_hbm, edst_hbm, p0_hbm, rin_hbm,
                agg_out, cacc_out,
                srcA, srcB, dstA, dstB, rows0_v, rows1_v, rv0_v, rv1_v, zb_v,
                acc_s, cacc_s, rin_s,
                gsem0, gsem1, vsem0, vsem1, isemA, isemB):
    c, s, w = _worker_id()
    pair = _mp_pipeline(esrc_hbm, edst_hbm, p0_hbm, acc_s,
                        srcA, srcB, dstA, dstB,
                        [rows0_v, rows1_v], [gsem0, gsem1], [isemA, isemB],
                        w, rin_s=rin_s, cacc_s=cacc_s,
                        rvs=[rv0_v, rv1_v], vsems=[vsem0, vsem1])
    for k in range(CH // L16):
        zb_v[pl.ds(k * L16, L16)] = jnp.zeros((L16,), F32)
    _zero_rows(rows0_v)
    _zero_stripe_2d(rows0_v, acc_s, s)
    _zero_stripe_1d(zb_v, cacc_s, s)
    pltpu.sync_copy(rin_hbm.at[pl.ds(s * RPS, RPS)], rin_s.at[pl.ds(s * RPS, RPS)])
    pltpu.make_async_copy(esrc_hbm.at[pl.ds(0, QG)], srcA, isemA).wait()
    pltpu.make_async_copy(esrc_hbm.at[pl.ds(0, QG)], dstA, isemA).wait()
    pltpu.async_copy(p0_hbm.at[srcA.at[0, 0]], rows0_v, gsem0)
    plsc.subcore_barrier()
    pltpu.async_copy(rin_s.at[dstA.at[0, 0]], rv0_v, vsem0)
    lax.fori_loop(0, NG // 2, pair, 0)
    plsc.subcore_barrier()
    row0 = c * NP + s * RPS
    pltpu.sync_copy(acc_s.at[pl.ds(s * RPS, RPS)], agg_out.at[pl.ds(row0, RPS)])
    pltpu.sync_copy(cacc_s.at[pl.ds(s * RPS, RPS)], cacc_out.at[pl.ds(row0, RPS)])


def _pass3_body(esrc_hbm, edst_hbm, p1_hbm,
                agg_out,
                srcA, srcB, dstA, dstB, rows0_v, rows1_v,
                acc_s,
                gsem0, gsem1, isemA, isemB):
    c, s, w = _worker_id()
    pair = _mp_pipeline(esrc_hbm, edst_hbm, p1_hbm, acc_s,
                        srcA, srcB, dstA, dstB,
                        [rows0_v, rows1_v], [gsem0, gsem1], [isemA, isemB], w)
    _zero_rows(rows0_v)
    _zero_stripe_2d(rows0_v, acc_s, s)
    pltpu.make_async_copy(esrc_hbm.at[pl.ds(0, QG)], srcA, isemA).wait()
    pltpu.make_async_copy(esrc_hbm.at[pl.ds(0, QG)], dstA, isemA).wait()
    pltpu.async_copy(p1_hbm.at[srcA.at[0, 0]], rows0_v, gsem0)
    plsc.subcore_barrier()
    lax.fori_loop(0, NG // 2, pair, 0)
    plsc.subcore_barrier()
    row0 = c * NP + s * RPS
    pltpu.sync_copy(acc_s.at[pl.ds(s * RPS, RPS)], agg_out.at[pl.ds(row0, RPS)])


def _sc_mesh():
    return plsc.VectorSubcoreMesh(core_axis_name="c", subcore_axis_name="s",
                                  num_cores=NC, num_subcores=NS)


def _tc_a_body(h0p, degop, degip, p0_ref, rin_ref, rout_ref):
    dego = degop[...][:NP] + degop[...][NP:]
    degi = degip[...][:NP] + degip[...][NP:]
    rout = lax.rsqrt(jnp.maximum(dego, 1.0))
    rin = lax.rsqrt(jnp.maximum(degi, 1.0))
    rin_ref[...] = rin
    rout_ref[...] = rout
    p0_ref[...] = h0p[...] * rout[:, None]


def _tc_b_body(aggp, rin, rout, p1_ref):
    agg = aggp[...][:NP] + aggp[...][NP:]
    h1 = jnp.maximum(agg * rin[...][:, None], 0.0)
    p1_ref[...] = h1 * rout[...][:, None]


def _tc_c_body(aggp, rin, rout, caccp, w1, w2, out_ref):
    agg = aggp[...][:NP] + aggp[...][NP:]
    a1 = agg * rin[...][:, None]
    h2 = jnp.maximum(jnp.dot(a1, w1[...], preferred_element_type=F32), 0.0)
    cc = (caccp[...][:NP] + caccp[...][NP:]) * rout[...]
    rid = lax.broadcasted_iota(I32, (NP,), 0)
    wvec = jnp.where(rid < N_NODES, cc, 0.0)
    s = jnp.sum(h2 * wvec[:, None], axis=0, keepdims=True)
    out_ref[...] = jnp.dot(s, w2[...], preferred_element_type=F32)


def kernel(feature, edge_index, table, W1, W2):
    n = feature.shape[0]
    nf = feature.shape[1]
    src = edge_index[0].astype(I32)
    dst = edge_index[1].astype(I32)

    e_pad = E_PAD - src.shape[0]
    pad_rows = N_NODES + (jnp.arange(e_pad, dtype=I32) % N_DUMMY)
    srcp = jnp.concatenate([src, pad_rows]).reshape(NW * CHUNKS_E, 1, CH)
    dstp = jnp.concatenate([dst, pad_rows]).reshape(NW * CHUNKS_E, 1, CH)

    gidx = feature.reshape(-1).astype(I32)
    nidx = jnp.repeat(jnp.arange(n, dtype=I32), nf)
    g_pad = G_PAD - gidx.shape[0]
    gpad_rows = jnp.arange(g_pad, dtype=I32) % jnp.int32(table.shape[0])
    npad_rows = N_NODES + (jnp.arange(g_pad, dtype=I32) % N_DUMMY)
    gidxp = jnp.concatenate([gidx, gpad_rows]).reshape(NW * CHUNKS_G, 1, CH)
    nidxp = jnp.concatenate([nidx, npad_rows]).reshape(NW * CHUNKS_G, 1, CH)

    mesh = _sc_mesh()

    pass1 = pl.kernel(
        _pass1_body, mesh=mesh,
        out_type=[jax.ShapeDtypeStruct((NP, D), F32),
                  jax.ShapeDtypeStruct((2 * NP,), F32),
                  jax.ShapeDtypeStruct((2 * NP,), F32)],
        scratch_types=[
            pltpu.VMEM((CHUNKS_G, 1, CH), I32),
            pltpu.VMEM((CHUNKS_G, 1, CH), I32),
            pltpu.VMEM((QG, 1, CH), I32),
            pltpu.VMEM((QG, 1, CH), I32),
            pltpu.VMEM((QG, 1, CH), I32),
            pltpu.VMEM((QG, 1, CH), I32),
            pltpu.VMEM((CH, D), F32),
            pltpu.VMEM((CH, D), F32),
            pltpu.VMEM((CH,), F32),
            pltpu.VMEM_SHARED((NP, D), F32),
            pltpu.VMEM_SHARED((NP,), F32),
            pltpu.VMEM_SHARED((NP,), F32),
            pltpu.SemaphoreType.DMA,
            pltpu.SemaphoreType.DMA,
            pltpu.SemaphoreType.DMA,
            pltpu.SemaphoreType.DMA,
            pltpu.SemaphoreType.DMA,
            pltpu.SemaphoreType.DMA,
        ])
    h0p, degop, degip = pass1(gidxp, nidxp, srcp, dstp, table)

    tc_a = pl.pallas_call(
        _tc_a_body,
        out_shape=[jax.ShapeDtypeStruct((NP, D), F32),
                   jax.ShapeDtypeStruct((NP,), F32),
                   jax.ShapeDtypeStruct((NP,), F32)])
    p0, rin, rout = tc_a(h0p, degop, degip)

    pass2 = pl.kernel(
        _pass2_body, mesh=mesh,
        out_type=[jax.ShapeDtypeStruct((2 * NP, D), F32),
                  jax.ShapeDtypeStruct((2 * NP,), F32)],
        scratch_types=[
            pltpu.VMEM((QG, 1, CH), I32),
            pltpu.VMEM((QG, 1, CH), I32),
            pltpu.VMEM((QG, 1, CH), I32),
            pltpu.VMEM((QG, 1, CH), I32),
            pltpu.VMEM((CH, D), F32),
            pltpu.VMEM((CH, D), F32),
            pltpu.VMEM((CH,), F32),
            pltpu.VMEM((CH,), F32),
            pltpu.VMEM((CH,), F32),
            pltpu.VMEM_SHARED((NP, D), F32),
            pltpu.VMEM_SHARED((NP,), F32),
            pltpu.VMEM_SHARED((NP,), F32),
            pltpu.SemaphoreType.DMA,
            pltpu.SemaphoreType.DMA,
            pltpu.SemaphoreType.DMA,
            pltpu.SemaphoreType.DMA,
            pltpu.SemaphoreType.DMA,
            pltpu.SemaphoreType.DMA,
        ])
    agg0p, caccp = pass2(srcp, dstp, p0, rin)

    tc_b = pl.pallas_call(
        _tc_b_body,
        out_shape=jax.ShapeDtypeStruct((NP, D), F32))
    p1 = tc_b(agg0p, rin, rout)

    pass3 = pl.kernel(
        _pass3_body, mesh=mesh,
        out_type=jax.ShapeDtypeStruct((2 * NP, D), F32),
        scratch_types=[
            pltpu.VMEM((QG, 1, CH), I32),
            pltpu.VMEM((QG, 1, CH), I32),
            pltpu.VMEM((QG, 1, CH), I32),
            pltpu.VMEM((QG, 1, CH), I32),
            pltpu.VMEM((CH, D), F32),
            pltpu.VMEM((CH, D), F32),
            pltpu.VMEM_SHARED((NP, D), F32),
            pltpu.SemaphoreType.DMA,
            pltpu.SemaphoreType.DMA,
            pltpu.SemaphoreType.DMA,
            pltpu.SemaphoreType.DMA,
        ])
    agg1p = pass3(srcp, dstp, p1)

    tc_c = pl.pallas_call(
        _tc_c_body,
        out_shape=jax.ShapeDtypeStruct((1, D), F32))
    out = tc_c(agg1p, rin, rout, caccp, W1, W2)
    return out.reshape(D)


# speedup vs baseline: 1.0356x; 1.0071x over previous
import jax
import jax.numpy as jnp
from jax import lax
from jax.experimental import pallas as pl
from jax.experimental.pallas import tpu as pltpu
from jax.experimental.pallas import tpu_sc as plsc

F32 = jnp.float32
I32 = jnp.int32

D = 128
L16 = 16
NC = 2
NS = 16
NW = NC * NS
CH = 128

N_NODES = 10000
NP = 10240
RPS = NP // NS
N_DUMMY = NP - N_NODES

N_EDGES = 320000
CHUNKS_E = 80
EPT_E = CHUNKS_E * CH
E_PAD = EPT_E * NW

N_EMB = N_NODES * 8
CHUNKS_G = 20
EPT_G = CHUNKS_G * CH
G_PAD = EPT_G * NW

QG = 8
NG = CHUNKS_E // QG


def _worker_id():
    c = lax.axis_index("c")
    s = lax.axis_index("s")
    return c, s, c * NS + s


def _zero_rows(rows_v):
    def body(i, carry):
        for k in range(D // L16):
            rows_v[i, pl.ds(k * L16, L16)] = jnp.zeros((L16,), F32)
        return carry
    lax.fori_loop(0, CH, body, 0)


def _zero_stripe_2d(zsrc_v, acc_s, s, sem):
    for j in range(RPS // CH):
        pltpu.async_copy(zsrc_v, acc_s.at[pl.ds(s * RPS + j * CH, CH)], sem)


def _zero_drain_2d(zsrc_v, acc_s, sem):
    for j in range(RPS // CH):
        pltpu.make_async_copy(zsrc_v, acc_s.at[pl.ds(0, CH)], sem).wait()


def _zero_stripe_1d(zsrc_row, acc_s, s, sem):
    for j in range(RPS // CH):
        pltpu.async_copy(zsrc_row, acc_s.at[pl.ds(s * RPS + j * CH, CH)], sem)


def _zero_drain_1d(zsrc_row, acc_s, sem):
    for j in range(RPS // CH):
        pltpu.make_async_copy(zsrc_row, acc_s.at[pl.ds(0, CH)], sem).wait()


def _pass1_body(gidx_hbm, nidx_hbm, esrc_hbm, edst_hbm, table_hbm,
                h0_out, dego_out, degi_out,
                gidx_v, nidx_v, srcA, srcB, dstA, dstB, rows0_v, rows1_v,
                ones_v,
                acc_s, dego_s, degi_s, sem0, sem1, dsem, isemA, isemB, isem):
    c, s, w = _worker_id()
    ebase = w * CHUNKS_E

    def ld_deg(gi, S, Dx, isem_):
        nb = ebase + gi * QG
        pltpu.async_copy(esrc_hbm.at[pl.ds(nb, QG)], S, isem_)
        pltpu.async_copy(edst_hbm.at[pl.ds(nb, QG)], Dx, isem_)

    cps = [pltpu.async_copy(gidx_hbm.at[pl.ds(w * CHUNKS_G, CHUNKS_G)], gidx_v, isem),
           pltpu.async_copy(nidx_hbm.at[pl.ds(w * CHUNKS_G, CHUNKS_G)], nidx_v, isem)]
    ld_deg(0, srcA, dstA, isemA)
    ld_deg(1, srcB, dstB, isemB)
    for k in range(CH // L16):
        ones_v[pl.ds(k * L16, L16)] = jnp.ones((L16,), F32)
    _zero_rows(rows0_v)
    _zero_stripe_2d(rows0_v, acc_s, s, sem0)
    _zero_stripe_1d(rows0_v.at[0], dego_s, s, sem0)
    _zero_stripe_1d(rows0_v.at[0], degi_s, s, sem0)
    for cp in cps:
        cp.wait()
    _zero_drain_2d(rows0_v, acc_s, sem0)
    _zero_drain_1d(rows0_v.at[0], dego_s, sem0)
    _zero_drain_1d(rows0_v.at[0], degi_s, sem0)
    plsc.subcore_barrier()

    rows = [rows0_v, rows1_v]
    sems = [sem0, sem1]

    def g_start(i, b):
        pltpu.async_copy(table_hbm.at[gidx_v.at[i, 0]], rows[b], sems[b])

    def g_wait(b):
        pltpu.make_async_copy(table_hbm.at[pl.ds(0, CH)], rows[b], sems[b]).wait()

    def scat(i, b):
        pltpu.sync_copy(rows[b], acc_s.at[nidx_v.at[i, 0]], add=True)

    def deg_group(gi, S, Dx, isemP, isemQ):
        pltpu.make_async_copy(esrc_hbm.at[pl.ds(0, QG)], S, isemP).wait()
        pltpu.make_async_copy(esrc_hbm.at[pl.ds(0, QG)], Dx, isemP).wait()
        for j in range(QG):
            pltpu.async_copy(ones_v, dego_s.at[S.at[j, 0]], dsem, add=True)
            pltpu.async_copy(ones_v, degi_s.at[Dx.at[j, 0]], dsem, add=True)

    def deg_drain_reload(gi, S, Dx, isemP):
        def dr(i, carry):
            pltpu.make_async_copy(ones_v, dego_s.at[S.at[0, 0]], dsem).wait()
            pltpu.make_async_copy(ones_v, degi_s.at[Dx.at[0, 0]], dsem).wait()
            return carry
        lax.fori_loop(0, QG, dr, 0)

        @pl.when(gi + 2 < NG)
        def _():
            ld_deg(gi + 2, S, Dx, isemP)

    g_start(0, 0)

    def pair(g, carry):
        i0 = 2 * g
        g_start(i0 + 1, 1)
        degA = (g % 2) == 0

        @pl.when(degA)
        def _():
            deg_group(g, srcA, dstA, isemA, isemB)

        @pl.when(jnp.logical_not(degA))
        def _():
            deg_group(g, srcB, dstB, isemB, isemA)
        g_wait(0)
        scat(i0, 0)

        @pl.when(i0 + 2 < CHUNKS_G)
        def _():
            g_start(i0 + 2, 0)

        @pl.when(degA)
        def _():
            deg_drain_reload(g, srcA, dstA, isemA)

        @pl.when(jnp.logical_not(degA))
        def _():
            deg_drain_reload(g, srcB, dstB, isemB)
        g_wait(1)
        scat(i0 + 1, 1)
        return carry
    lax.fori_loop(0, CHUNKS_G // 2, pair, 0)

    plsc.subcore_barrier()
    row0 = c * NP + s * RPS
    hrow = c * (NP // 2) + s * (RPS // 2)
    pltpu.sync_copy(acc_s.at[pl.ds(hrow, RPS // 2)], h0_out.at[pl.ds(hrow, RPS // 2)])
    pltpu.sync_copy(dego_s.at[pl.ds(s * RPS, RPS)], dego_out.at[pl.ds(row0, RPS)])
    pltpu.sync_copy(degi_s.at[pl.ds(s * RPS, RPS)], degi_out.at[pl.ds(row0, RPS)])


def _mp_pipeline(esrc_hbm, edst_hbm, p_hbm, acc_s,
                 srcA, srcB, dstA, dstB, rows, gsems, isems,
                 w, rin_s=None, cacc_s=None, rvs=None, vsems=None):
    base = w * CHUNKS_E
    with_c = rin_s is not None

    def ld_group(gi, S, Dx, isem):
        nb = base + gi * QG
        pltpu.async_copy(esrc_hbm.at[pl.ds(nb, QG)], S, isem)
        pltpu.async_copy(edst_hbm.at[pl.ds(nb, QG)], Dx, isem)

    def ld_wait(isem):
        pltpu.make_async_copy(esrc_hbm.at[pl.ds(0, QG)], srcA, isem).wait()
        pltpu.make_async_copy(esrc_hbm.at[pl.ds(0, QG)], dstA, isem).wait()

    def g_start(S, Dx, j, b):
        pltpu.async_copy(p_hbm.at[S.at[j, 0]], rows[b], gsems[b])
        if with_c:
            pltpu.async_copy(rin_s.at[Dx.at[j, 0]], rvs[b], vsems[b])

    def g_wait(b):
        pltpu.make_async_copy(p_hbm.at[pl.ds(0, CH)], rows[b], gsems[b]).wait()
        if with_c:
            pltpu.make_async_copy(rin_s.at[pl.ds(0, CH)], rvs[b], vsems[b]).wait()

    def scat(S, Dx, j, b):
        pltpu.sync_copy(rows[b], acc_s.at[Dx.at[j, 0]], add=True)
        if with_c:
            pltpu.sync_copy(rvs[b], cacc_s.at[S.at[j, 0]], add=True)

    def group(gi, S, Dx, isemP, So, Do, isemQ):
        for j in range(QG):
            b = j % 2
            if j + 1 < QG:
                g_start(S, Dx, j + 1, 1 - b)
            else:
                @pl.when(gi + 1 < NG)
                def _():
                    ld_wait(isemQ)
                    g_start(So, Do, 0, 1 - b)
            g_wait(b)
            scat(S, Dx, j, b)

        @pl.when(gi + 2 < NG)
        def _():
            ld_group(gi + 2, S, Dx, isemP)

    def pair(g, carry):
        group(2 * g, srcA, dstA, isems[0], srcB, dstB, isems[1])
        group(2 * g + 1, srcB, dstB, isems[1], srcA, dstA, isems[0])
        return carry

    ld_group(0, srcA, dstA, isems[0])
    ld_group(1, srcB, dstB, isems[1])
    return pair


def _pass2_body(esrc_hbm, edst_hbm, p0_hbm, rin_hbm,
                agg_out, cacc_out,
                srcA, srcB, dstA, dstB, rows0_v, rows1_v, rv0_v, rv1_v, zb_v,
                acc_s, cacc_s, rin_s,
                gsem0, gsem1, vsem0, vsem1, isemA, isemB):
    c, s, w = _worker_id()
    pair = _mp_pipeline(esrc_hbm, edst_hbm, p0_hbm, acc_s,
                        srcA, srcB, dstA, dstB,
                        [rows0_v, rows1_v], [gsem0, gsem1], [isemA, isemB],
                        w, rin_s=rin_s, cacc_s=cacc_s,
                        rvs=[rv0_v, rv1_v], vsems=[vsem0, vsem1])
    for k in range(CH // L16):
        zb_v[pl.ds(k * L16, L16)] = jnp.zeros((L16,), F32)
    _zero_rows(rows0_v)
    _zero_stripe_2d(rows0_v, acc_s, s, gsem0)
    _zero_stripe_1d(zb_v, cacc_s, s, gsem0)
    pltpu.async_copy(rin_hbm.at[pl.ds(s * RPS, RPS)], rin_s.at[pl.ds(s * RPS, RPS)], gsem1)
    pltpu.make_async_copy(esrc_hbm.at[pl.ds(0, QG)], srcA, isemA).wait()
    pltpu.make_async_copy(esrc_hbm.at[pl.ds(0, QG)], dstA, isemA).wait()
    _zero_drain_2d(rows0_v, acc_s, gsem0)
    _zero_drain_1d(zb_v, cacc_s, gsem0)
    pltpu.make_async_copy(rin_hbm.at[pl.ds(0, RPS)], rin_s.at[pl.ds(0, RPS)], gsem1).wait()
    pltpu.async_copy(p0_hbm.at[srcA.at[0, 0]], rows0_v, gsem0)
    plsc.subcore_barrier()
    pltpu.async_copy(rin_s.at[dstA.at[0, 0]], rv0_v, vsem0)
    lax.fori_loop(0, NG // 2, pair, 0)
    plsc.subcore_barrier()
    row0 = c * NP + s * RPS
    pltpu.sync_copy(acc_s.at[pl.ds(s * RPS, RPS)], agg_out.at[pl.ds(row0, RPS)])
    pltpu.sync_copy(cacc_s.at[pl.ds(s * RPS, RPS)], cacc_out.at[pl.ds(row0, RPS)])


def _pass3_body(esrc_hbm, edst_hbm, p1_hbm,
                agg_out,
                srcA, srcB, dstA, dstB, rows0_v, rows1_v,
                acc_s,
                gsem0, gsem1, isemA, isemB):
    c, s, w = _worker_id()
    pair = _mp_pipeline(esrc_hbm, edst_hbm, p1_hbm, acc_s,
                        srcA, srcB, dstA, dstB,
                        [rows0_v, rows1_v], [gsem0, gsem1], [isemA, isemB], w)
    _zero_rows(rows0_v)
    _zero_stripe_2d(rows0_v, acc_s, s, gsem0)
    pltpu.make_async_copy(esrc_hbm.at[pl.ds(0, QG)], srcA, isemA).wait()
    pltpu.make_async_copy(esrc_hbm.at[pl.ds(0, QG)], dstA, isemA).wait()
    _zero_drain_2d(rows0_v, acc_s, gsem0)
    pltpu.async_copy(p1_hbm.at[srcA.at[0, 0]], rows0_v, gsem0)
    plsc.subcore_barrier()
    lax.fori_loop(0, NG // 2, pair, 0)
    plsc.subcore_barrier()
    row0 = c * NP + s * RPS
    pltpu.sync_copy(acc_s.at[pl.ds(s * RPS, RPS)], agg_out.at[pl.ds(row0, RPS)])


def _sc_mesh():
    return plsc.VectorSubcoreMesh(core_axis_name="c", subcore_axis_name="s",
                                  num_cores=NC, num_subcores=NS)


def _tc_a_body(h0p, degop, degip, p0_ref, rin_ref, rout_ref):
    dego = degop[...][:NP] + degop[...][NP:]
    degi = degip[...][:NP] + degip[...][NP:]
    rout = lax.rsqrt(jnp.maximum(dego, 1.0))
    rin = lax.rsqrt(jnp.maximum(degi, 1.0))
    rin_ref[...] = rin
    rout_ref[...] = rout
    p0_ref[...] = h0p[...] * rout[:, None]


def _tc_b_body(aggp, rin, rout, p1_ref):
    agg = aggp[...][:NP] + aggp[...][NP:]
    h1 = jnp.maximum(agg * rin[...][:, None], 0.0)
    p1_ref[...] = h1 * rout[...][:, None]


def _tc_c_body(aggp, rin, rout, caccp, w1, w2, out_ref):
    agg = aggp[...][:NP] + aggp[...][NP:]
    a1 = agg * rin[...][:, None]
    h2 = jnp.maximum(jnp.dot(a1, w1[...], preferred_element_type=F32), 0.0)
    cc = (caccp[...][:NP] + caccp[...][NP:]) * rout[...]
    rid = lax.broadcasted_iota(I32, (NP,), 0)
    wvec = jnp.where(rid < N_NODES, cc, 0.0)
    s = jnp.sum(h2 * wvec[:, None], axis=0, keepdims=True)
    out_ref[...] = jnp.dot(s, w2[...], preferred_element_type=F32)


def kernel(feature, edge_index, table, W1, W2):
    n = feature.shape[0]
    nf = feature.shape[1]
    src = edge_index[0].astype(I32)
    dst = edge_index[1].astype(I32)

    e_pad = E_PAD - src.shape[0]
    pad_rows = N_NODES + (jnp.arange(e_pad, dtype=I32) % N_DUMMY)
    srcp = jnp.concatenate([src, pad_rows]).reshape(NW * CHUNKS_E, 1, CH)
    dstp = jnp.concatenate([dst, pad_rows]).reshape(NW * CHUNKS_E, 1, CH)

    gidx = feature.reshape(-1).astype(I32)
    nidx = jnp.repeat(jnp.arange(n, dtype=I32), nf)
    g_pad = G_PAD - gidx.shape[0]
    gpad_rows = jnp.arange(g_pad, dtype=I32) % jnp.int32(table.shape[0])
    npad_rows = N_NODES + (jnp.arange(g_pad, dtype=I32) % N_DUMMY)
    gidxp = jnp.concatenate([gidx, gpad_rows]).reshape(NW * CHUNKS_G, 1, CH)
    nidxp = jnp.concatenate([nidx, npad_rows]).reshape(NW * CHUNKS_G, 1, CH)

    mesh = _sc_mesh()

    pass1 = pl.kernel(
        _pass1_body, mesh=mesh,
        out_type=[jax.ShapeDtypeStruct((NP, D), F32),
                  jax.ShapeDtypeStruct((2 * NP,), F32),
                  jax.ShapeDtypeStruct((2 * NP,), F32)],
        scratch_types=[
            pltpu.VMEM((CHUNKS_G, 1, CH), I32),
            pltpu.VMEM((CHUNKS_G, 1, CH), I32),
            pltpu.VMEM((QG, 1, CH), I32),
            pltpu.VMEM((QG, 1, CH), I32),
            pltpu.VMEM((QG, 1, CH), I32),
            pltpu.VMEM((QG, 1, CH), I32),
            pltpu.VMEM((CH, D), F32),
            pltpu.VMEM((CH, D), F32),
            pltpu.VMEM((CH,), F32),
            pltpu.VMEM_SHARED((NP, D), F32),
            pltpu.VMEM_SHARED((NP,), F32),
            pltpu.VMEM_SHARED((NP,), F32),
            pltpu.SemaphoreType.DMA,
            pltpu.SemaphoreType.DMA,
            pltpu.SemaphoreType.DMA,
            pltpu.SemaphoreType.DMA,
            pltpu.SemaphoreType.DMA,
            pltpu.SemaphoreType.DMA,
        ])
    h0p, degop, degip = pass1(gidxp, nidxp, srcp, dstp, table)

    tc_a = pl.pallas_call(
        _tc_a_body,
        out_shape=[jax.ShapeDtypeStruct((NP, D), F32),
                   jax.ShapeDtypeStruct((NP,), F32),
                   jax.ShapeDtypeStruct((NP,), F32)])
    p0, rin, rout = tc_a(h0p, degop, degip)

    pass2 = pl.kernel(
        _pass2_body, mesh=mesh,
        out_type=[jax.ShapeDtypeStruct((2 * NP, D), F32),
                  jax.ShapeDtypeStruct((2 * NP,), F32)],
        scratch_types=[
            pltpu.VMEM((QG, 1, CH), I32),
            pltpu.VMEM((QG, 1, CH), I32),
            pltpu.VMEM((QG, 1, CH), I32),
            pltpu.VMEM((QG, 1, CH), I32),
            pltpu.VMEM((CH, D), F32),
            pltpu.VMEM((CH, D), F32),
            pltpu.VMEM((CH,), F32),
            pltpu.VMEM((CH,), F32),
            pltpu.VMEM((CH,), F32),
            pltpu.VMEM_SHARED((NP, D), F32),
            pltpu.VMEM_SHARED((NP,), F32),
            pltpu.VMEM_SHARED((NP,), F32),
            pltpu.SemaphoreType.DMA,
            pltpu.SemaphoreType.DMA,
            pltpu.SemaphoreType.DMA,
            pltpu.SemaphoreType.DMA,
            pltpu.SemaphoreType.DMA,
            pltpu.SemaphoreType.DMA,
        ])
    agg0p, caccp = pass2(srcp, dstp, p0, rin)

    tc_b = pl.pallas_call(
        _tc_b_body,
        out_shape=jax.ShapeDtypeStruct((NP, D), F32))
    p1 = tc_b(agg0p, rin, rout)

    pass3 = pl.kernel(
        _pass3_body, mesh=mesh,
        out_type=jax.ShapeDtypeStruct((2 * NP, D), F32),
        scratch_types=[
            pltpu.VMEM((QG, 1, CH), I32),
            pltpu.VMEM((QG, 1, CH), I32),
            pltpu.VMEM((QG, 1, CH), I32),
            pltpu.VMEM((QG, 1, CH), I32),
            pltpu.VMEM((CH, D), F32),
            pltpu.VMEM((CH, D), F32),
            pltpu.VMEM_SHARED((NP, D), F32),
            pltpu.SemaphoreType.DMA,
            pltpu.SemaphoreType.DMA,
            pltpu.SemaphoreType.DMA,
            pltpu.SemaphoreType.DMA,
        ])
    agg1p = pass3(srcp, dstp, p1)

    tc_c = pl.pallas_call(
        _tc_c_body,
        out_shape=jax.ShapeDtypeStruct((1, D), F32))
    out = tc_c(agg1p, rin, rout, caccp, W1, W2)
    return out.reshape(D)
